# baseline (speedup 1.0000x reference)
import functools

import jax
import jax.numpy as jnp
from jax import lax
from jax.experimental import pallas as pl
from jax.experimental.pallas import tpu as pltpu
from jax.experimental.pallas import tpu_sc as plsc

N = 10000
E = 320000
D = 128
H = 8
HD = D // H
DH = D // 2
MW = DH + 2 * H

NC = 2
NS = 16
C = 128
NCHUNK = E // C
SLOTS = 158

ROWS_PER_SUB = 624
TAIL0 = NS * ROWS_PER_SUB
TAIL = N - TAIL0
ACC_ROWS = N + 8
ZR = 104


def _prep_body(x_ref, w_ref, a1_ref, a2_ref, h_ref, s2_ref, d2_ref):
    h = jnp.dot(x_ref[...], w_ref[...], preferred_element_type=jnp.float32)
    h_ref[...] = h
    s2_ref[...] = jnp.dot(h, a1_ref[...], preferred_element_type=jnp.float32)
    d2_ref[...] = jnp.dot(h, a2_ref[...], preferred_element_type=jnp.float32)


def _tc_prep(x, W, A_src2, A_dst2):
    blk = 2000
    grid = N // blk
    return pl.pallas_call(
        _prep_body,
        grid=(grid,),
        in_specs=[
            pl.BlockSpec((blk, D), lambda i: (i, 0)),
            pl.BlockSpec((D, D), lambda i: (0, 0)),
            pl.BlockSpec((D, 2 * H), lambda i: (0, 0)),
            pl.BlockSpec((D, 2 * H), lambda i: (0, 0)),
        ],
        out_specs=[
            pl.BlockSpec((blk, D), lambda i: (i, 0)),
            pl.BlockSpec((blk, 2 * H), lambda i: (i, 0)),
            pl.BlockSpec((blk, 2 * H), lambda i: (i, 0)),
        ],
        out_shape=[
            jax.ShapeDtypeStruct((N, D), jnp.float32),
            jax.ShapeDtypeStruct((N, 2 * H), jnp.float32),
            jax.ShapeDtypeStruct((N, 2 * H), jnp.float32),
        ],
    )(x, W, A_src2, A_dst2)


_GDN = lax.GatherDimensionNumbers(
    offset_dims=(), collapsed_slice_dims=(0,), start_index_map=(0,))


def _lane_bcast(vec, idx):
    return lax.gather(vec, idx[:, None], _GDN, (1,),
                      mode=lax.GatherScatterMode.PROMISE_IN_BOUNDS)


def _sc_body(h2_hbm, d2_hbm, src_hbm, dst_hbm, acc_out,
             srcA, dstA, dsA, d2A, rowsA, msgA, isemA, gsemA, ssemA,
             srcB, dstB, dsB, d2B, rowsB, msgB, isemB, gsemB, ssemB,
             zb_v, acc_sh):
    core = lax.axis_index("c")
    sub = lax.axis_index("s")
    row0 = sub * ROWS_PER_SUB

    A = (srcA, dstA, dsA, None, d2A, rowsA, msgA, isemA, gsemA, ssemA)
    B = (srcB, dstB, dsB, None, d2B, rowsB, msgB, isemB, gsemB, ssemB)

    zero16 = jnp.zeros((16,), jnp.float32)

    def zrow(r, _):
        for k in range(MW // 16):
            zb_v[r, pl.ds(k * 16, 16)] = zero16
        return 0

    lax.fori_loop(0, ZR, zrow, 0)
    for b in range(ROWS_PER_SUB // ZR):
        pltpu.sync_copy(zb_v, acc_sh.at[pl.ds(row0 + b * ZR, ZR)])

    @pl.when(sub == NS - 1)
    def _zero_tail():
        pltpu.sync_copy(zb_v.at[pl.ds(0, ACC_ROWS - TAIL0)],
                        acc_sh.at[pl.ds(TAIL0, ACC_ROWS - TAIL0)])

    plsc.subcore_barrier()

    head_idx = [jnp.full((16,), 0, jnp.int32) + (core * (H // NC) + hd)
                for hd in range(H // NC)]

    def _valid01(chunk):
        return lax.shift_right_logical(chunk - NCHUNK, 31)

    def idx_issue(s, X):
        chunk = sub + s * NS
        base = chunk * _valid01(chunk) * C
        pltpu.make_async_copy(src_hbm.at[pl.ds(base, C)], X[0], X[7]).start()
        pltpu.make_async_copy(dst_hbm.at[pl.ds(base, C)], X[1], X[7]).start()

    def idx_wait(X):
        pltpu.make_async_copy(src_hbm.at[pl.ds(0, C)], X[0], X[7]).wait()
        pltpu.make_async_copy(dst_hbm.at[pl.ds(0, C)], X[1], X[7]).wait()

    def g_issue(X):
        pltpu.make_async_copy(d2_hbm.at[X[1]], X[4], X[8]).start()
        pltpu.make_async_copy(h2_hbm.at[core].at[X[0]], X[5], X[8]).start()

    def g_wait(X):
        pltpu.make_async_copy(d2_hbm.at[X[1]], X[4], X[8]).wait()
        pltpu.make_async_copy(h2_hbm.at[core].at[X[0]], X[5], X[8]).wait()

    def sc_issue(X):
        pltpu.make_async_copy(X[6], acc_sh.at[X[2]], X[9]).start(add=True)

    def sc_wait(X):
        pltpu.make_async_copy(X[6], acc_sh.at[X[2]], X[9]).wait()

    def dsfill(s, X):
        chunk = sub + s * NS
        vs = jnp.full((16,), 0, jnp.int32) + _valid01(chunk)
        iv = 1 - vs
        for g in range(C // 16):
            d16 = X[1][pl.ds(g * 16, 16)]
            X[2][pl.ds(g * 16, 16)] = d16 * vs + (N + (d16 & 7)) * iv

    def compute(X):
        d2_v, rows_v, msg_v = X[4], X[5], X[6]

        @plsc.parallel_loop(0, C, step=1, unroll=8)
        def edge(c):
            s2f = plsc.bitcast(rows_v[c, pl.ds(DH, 32)], jnp.float32)
            e2 = s2f + d2_v[c, :]
            w2 = jnp.exp(jnp.maximum(e2, e2 * 0.2))
            msg_v[c, pl.ds(DH, 16)] = w2
            for g in range(H // NC // 2):
                hpair = rows_v[c, pl.ds(g * 32, 32)]
                ha, hb = plsc.unpack(hpair, format=plsc.PackFormat.INTERLEAVED)
                wa = _lane_bcast(w2, head_idx[2 * g])
                wb = _lane_bcast(w2, head_idx[2 * g + 1])
                msg_v[c, pl.ds(g * 32, 16)] = ha * wa
                msg_v[c, pl.ds(g * 32 + 16, 16)] = hb * wb

    def half(s, cur, nxt):
        g_wait(cur)

        @pl.when(s >= 2)
        def _():
            sc_wait(cur)

        dsfill(s, cur)
        idx_issue(s + 2, cur)
        idx_wait(nxt)
        g_issue(nxt)
        compute(cur)
        sc_issue(cur)

    idx_issue(0, A)
    idx_wait(A)
    g_issue(A)
    idx_issue(1, B)

    def pair(kp, _):
        s = 2 * kp
        half(s, A, B)
        half(s + 1, B, A)
        return 0

    lax.fori_loop(0, SLOTS // 2, pair, 0)

    g_wait(A)
    idx_wait(B)
    sc_wait(A)
    sc_wait(B)
    plsc.subcore_barrier()

    pltpu.sync_copy(acc_sh.at[pl.ds(row0, ROWS_PER_SUB)],
                    acc_out.at[core, pl.ds(row0, ROWS_PER_SUB)])

    @pl.when(sub == NS - 1)
    def _copy_tail():
        pltpu.sync_copy(acc_sh.at[pl.ds(TAIL0, TAIL)],
                        acc_out.at[core, pl.ds(TAIL0, TAIL)])


_sc_edge = functools.partial(
    pl.kernel,
    out_type=jax.ShapeDtypeStruct((NC, N, MW), jnp.float32),
    mesh=plsc.VectorSubcoreMesh(
        core_axis_name="c", subcore_axis_name="s",
        num_cores=NC, num_subcores=NS,
    ),
    compiler_params=pltpu.CompilerParams(use_tc_tiling_on_sc=False,
                                         needs_layout_passes=False),
    scratch_types=[
        pltpu.VMEM((C,), jnp.int32),
        pltpu.VMEM((C,), jnp.int32),
        pltpu.VMEM((C,), jnp.int32),
        pltpu.VMEM((C, 2 * H), jnp.float32),
        pltpu.VMEM((C, DH + 32), jnp.bfloat16),
        pltpu.VMEM((C, MW), jnp.float32),
        pltpu.SemaphoreType.DMA,
        pltpu.SemaphoreType.DMA,
        pltpu.SemaphoreType.DMA,
        pltpu.VMEM((C,), jnp.int32),
        pltpu.VMEM((C,), jnp.int32),
        pltpu.VMEM((C,), jnp.int32),
        pltpu.VMEM((C, 2 * H), jnp.float32),
        pltpu.VMEM((C, DH + 32), jnp.bfloat16),
        pltpu.VMEM((C, MW), jnp.float32),
        pltpu.SemaphoreType.DMA,
        pltpu.SemaphoreType.DMA,
        pltpu.SemaphoreType.DMA,
        pltpu.VMEM((ZR, MW), jnp.float32),
        pltpu.VMEM_SHARED((ACC_ROWS, MW), jnp.float32),
    ],
)(_sc_body)


def _fin_body(x_ref, h_ref, s2_ref, d2_ref, acc_ref, r_ref, b_ref, o_ref):
    e2 = s2_ref[...] + d2_ref[...]
    w2 = jnp.exp(jnp.maximum(e2, e2 * 0.2))
    wex = jnp.dot(w2, r_ref[...], preferred_element_type=jnp.float32)
    den = acc_ref[0, :, DH:] + w2
    denx = jnp.dot(den, r_ref[...], preferred_element_type=jnp.float32)
    accs = jnp.concatenate([acc_ref[0, :, :DH], acc_ref[1, :, :DH]], axis=-1)
    acc = accs + h_ref[...] * wex
    o_ref[...] = acc / denx + b_ref[...] + x_ref[...]


def _tc_finalize(x, h, s2, d2, acc, R, bias2):
    blk = 2000
    grid = N // blk
    return pl.pallas_call(
        _fin_body,
        grid=(grid,),
        in_specs=[
            pl.BlockSpec((blk, D), lambda i: (i, 0)),
            pl.BlockSpec((blk, D), lambda i: (i, 0)),
            pl.BlockSpec((blk, 2 * H), lambda i: (i, 0)),
            pl.BlockSpec((blk, 2 * H), lambda i: (i, 0)),
            pl.BlockSpec((NC, blk, MW), lambda i: (0, i, 0)),
            pl.BlockSpec((2 * H, D), lambda i: (0, 0)),
            pl.BlockSpec((1, D), lambda i: (0, 0)),
        ],
        out_specs=pl.BlockSpec((blk, D), lambda i: (i, 0)),
        out_shape=jax.ShapeDtypeStruct((N, D), jnp.float32),
    )(x, h, s2, d2, acc, R, bias2)


def kernel(x, edge_index, W, att_src, att_dst, bias):
    src = edge_index[0].astype(jnp.int32)
    dst = edge_index[1].astype(jnp.int32)

    i = jnp.arange(D)
    j = jnp.arange(2 * H)
    sel = (i[:, None] // HD) == (j[None, :] % H)
    A_src2 = jnp.where(sel, att_src.reshape(D)[:, None], 0.0)
    A_dst2 = jnp.where(sel, att_dst.reshape(D)[:, None], 0.0)
    R = jnp.where((j[:, None] < H) & ((i[None, :] // HD) == j[:, None]),
                  1.0, 0.0)

    h, s2, d2 = _tc_prep(x, W, A_src2, A_dst2)
    n = x.shape[0]
    hb = h.astype(jnp.bfloat16)
    s2b = lax.bitcast_convert_type(s2, jnp.bfloat16).reshape(n, 32)

    def _shuf(part):
        return (part.reshape(n, 2, 2, HD).transpose(0, 1, 3, 2)
                .reshape(n, DH))

    h2 = jnp.stack([jnp.concatenate([_shuf(hb[:, :DH]), s2b], axis=1),
                    jnp.concatenate([_shuf(hb[:, DH:]), s2b], axis=1)])
    acc = _sc_edge(h2, d2, src, dst)
    return _tc_finalize(x, h, s2, d2, acc, R, bias[None, :])


# speedup vs baseline: 1.0756x; 1.0756x over previous
import functools

import jax
import jax.numpy as jnp
from jax import lax
from jax.experimental import pallas as pl
from jax.experimental.pallas import tpu as pltpu
from jax.experimental.pallas import tpu_sc as plsc

N = 10000
E = 320000
D = 128
H = 8
HD = D // H
DH = D // 2
MW = DH + 2 * H

NC = 2
NS = 16
C = 128
NCHUNK = E // C
SLOTS = 158

ROWS_PER_SUB = 624
TAIL0 = NS * ROWS_PER_SUB
TAIL = N - TAIL0
ACC_ROWS = N + 8
ZR = 104


def _prep_body(x_ref, w_ref, a1_ref, a2_ref, h_ref, s2_ref, d2_ref):
    h = jnp.dot(x_ref[...], w_ref[...], preferred_element_type=jnp.float32)
    h_ref[...] = h
    s2_ref[...] = jnp.dot(h, a1_ref[...], preferred_element_type=jnp.float32)
    d2_ref[...] = jnp.dot(h, a2_ref[...], preferred_element_type=jnp.float32)


def _tc_prep(x, W, A_src2, A_dst2):
    blk = 2000
    grid = N // blk
    return pl.pallas_call(
        _prep_body,
        grid=(grid,),
        in_specs=[
            pl.BlockSpec((blk, D), lambda i: (i, 0)),
            pl.BlockSpec((D, D), lambda i: (0, 0)),
            pl.BlockSpec((D, 2 * H), lambda i: (0, 0)),
            pl.BlockSpec((D, 2 * H), lambda i: (0, 0)),
        ],
        out_specs=[
            pl.BlockSpec((blk, D), lambda i: (i, 0)),
            pl.BlockSpec((blk, 2 * H), lambda i: (i, 0)),
            pl.BlockSpec((blk, 2 * H), lambda i: (i, 0)),
        ],
        out_shape=[
            jax.ShapeDtypeStruct((N, D), jnp.float32),
            jax.ShapeDtypeStruct((N, 2 * H), jnp.float32),
            jax.ShapeDtypeStruct((N, 2 * H), jnp.float32),
        ],
    )(x, W, A_src2, A_dst2)


_GDN = lax.GatherDimensionNumbers(
    offset_dims=(), collapsed_slice_dims=(0,), start_index_map=(0,))


def _lane_bcast(vec, idx):
    return lax.gather(vec, idx[:, None], _GDN, (1,),
                      mode=lax.GatherScatterMode.PROMISE_IN_BOUNDS)


def _sc_body(h2_hbm, d2_hbm, src_hbm, dst_hbm, acc_out,
             srcA, dstA, dsA, d2A, rowsA, msgA, isemA, gsemA, ssemA,
             srcB, dstB, dsB, d2B, rowsB, msgB, isemB, gsemB, ssemB,
             zb_v, acc_sh):
    core = lax.axis_index("c")
    sub = lax.axis_index("s")
    row0 = sub * ROWS_PER_SUB

    A = (srcA, dstA, dsA, None, d2A, rowsA, msgA, isemA, gsemA, ssemA)
    B = (srcB, dstB, dsB, None, d2B, rowsB, msgB, isemB, gsemB, ssemB)

    zero16 = jnp.zeros((16,), jnp.float32)

    def zrow(r, _):
        for k in range(MW // 16):
            zb_v[r, pl.ds(k * 16, 16)] = zero16
        return 0

    lax.fori_loop(0, ZR, zrow, 0)
    for b in range(ROWS_PER_SUB // ZR):
        pltpu.sync_copy(zb_v, acc_sh.at[pl.ds(row0 + b * ZR, ZR)])

    @pl.when(sub == NS - 1)
    def _zero_tail():
        pltpu.sync_copy(zb_v.at[pl.ds(0, ACC_ROWS - TAIL0)],
                        acc_sh.at[pl.ds(TAIL0, ACC_ROWS - TAIL0)])

    plsc.subcore_barrier()

    head_idx = [jnp.full((16,), 0, jnp.int32) + (core * (H // NC) + hd)
                for hd in range(H // NC)]

    def _valid01(chunk):
        return lax.shift_right_logical(chunk - NCHUNK, 31)

    def idx_issue(s, X):
        chunk = sub + s * NS
        base = chunk * _valid01(chunk) * C
        pltpu.make_async_copy(src_hbm.at[pl.ds(base, C)], X[0], X[7]).start()
        pltpu.make_async_copy(dst_hbm.at[pl.ds(base, C)], X[1], X[7]).start()

    def idx_wait(X):
        pltpu.make_async_copy(src_hbm.at[pl.ds(0, C)], X[0], X[7]).wait()
        pltpu.make_async_copy(dst_hbm.at[pl.ds(0, C)], X[1], X[7]).wait()

    def g_issue(X):
        pltpu.make_async_copy(d2_hbm.at[X[1]], X[4], X[8]).start()
        pltpu.make_async_copy(h2_hbm.at[core].at[X[0]], X[5], X[8]).start()

    def g_wait(X):
        pltpu.make_async_copy(d2_hbm.at[X[1]], X[4], X[8]).wait()
        pltpu.make_async_copy(h2_hbm.at[core].at[X[0]], X[5], X[8]).wait()

    def sc_issue(X):
        pltpu.make_async_copy(X[6], acc_sh.at[X[2]], X[9]).start(add=True)

    def sc_wait(X):
        pltpu.make_async_copy(X[6], acc_sh.at[X[2]], X[9]).wait()

    def dsfill(s, X):
        chunk = sub + s * NS
        vs = jnp.full((16,), 0, jnp.int32) + _valid01(chunk)
        iv = 1 - vs
        for g in range(C // 16):
            d16 = X[1][pl.ds(g * 16, 16)]
            X[2][pl.ds(g * 16, 16)] = d16 * vs + (N + (d16 & 7)) * iv

    def compute(X):
        d2_v, rows_v, msg_v = X[4], X[5], X[6]

        @plsc.parallel_loop(0, C, step=1, unroll=8)
        def edge(c):
            e2 = rows_v[c, pl.ds(DH, 16)] + d2_v[c, :]
            w2 = jnp.exp(jnp.maximum(e2, e2 * 0.2))
            msg_v[c, pl.ds(DH, 16)] = w2
            for hd in range(H // NC):
                ws = _lane_bcast(w2, head_idx[hd])
                msg_v[c, pl.ds(hd * 16, 16)] = rows_v[c, pl.ds(hd * 16, 16)] * ws

    def half(s, cur, nxt):
        g_wait(cur)

        @pl.when(s >= 2)
        def _():
            sc_wait(cur)

        dsfill(s, cur)
        idx_issue(s + 2, cur)
        idx_wait(nxt)
        g_issue(nxt)
        compute(cur)
        sc_issue(cur)

    idx_issue(0, A)
    idx_wait(A)
    g_issue(A)
    idx_issue(1, B)

    def pair(kp, _):
        s = 2 * kp
        half(s, A, B)
        half(s + 1, B, A)
        return 0

    lax.fori_loop(0, SLOTS // 2, pair, 0)

    g_wait(A)
    idx_wait(B)
    sc_wait(A)
    sc_wait(B)
    plsc.subcore_barrier()

    pltpu.sync_copy(acc_sh.at[pl.ds(row0, ROWS_PER_SUB)],
                    acc_out.at[core, pl.ds(row0, ROWS_PER_SUB)])

    @pl.when(sub == NS - 1)
    def _copy_tail():
        pltpu.sync_copy(acc_sh.at[pl.ds(TAIL0, TAIL)],
                        acc_out.at[core, pl.ds(TAIL0, TAIL)])


_sc_edge = functools.partial(
    pl.kernel,
    out_type=jax.ShapeDtypeStruct((NC, N, MW), jnp.float32),
    mesh=plsc.VectorSubcoreMesh(
        core_axis_name="c", subcore_axis_name="s",
        num_cores=NC, num_subcores=NS,
    ),
    compiler_params=pltpu.CompilerParams(use_tc_tiling_on_sc=False),
    scratch_types=[
        pltpu.VMEM((C,), jnp.int32),
        pltpu.VMEM((C,), jnp.int32),
        pltpu.VMEM((C,), jnp.int32),
        pltpu.VMEM((C, 2 * H), jnp.float32),
        pltpu.VMEM((C, MW), jnp.float32),
        pltpu.VMEM((C, MW), jnp.float32),
        pltpu.SemaphoreType.DMA,
        pltpu.SemaphoreType.DMA,
        pltpu.SemaphoreType.DMA,
        pltpu.VMEM((C,), jnp.int32),
        pltpu.VMEM((C,), jnp.int32),
        pltpu.VMEM((C,), jnp.int32),
        pltpu.VMEM((C, 2 * H), jnp.float32),
        pltpu.VMEM((C, MW), jnp.float32),
        pltpu.VMEM((C, MW), jnp.float32),
        pltpu.SemaphoreType.DMA,
        pltpu.SemaphoreType.DMA,
        pltpu.SemaphoreType.DMA,
        pltpu.VMEM((ZR, MW), jnp.float32),
        pltpu.VMEM_SHARED((ACC_ROWS, MW), jnp.float32),
    ],
)(_sc_body)


def _fin_body(x_ref, h_ref, s2_ref, d2_ref, acc_ref, r_ref, b_ref, o_ref):
    e2 = s2_ref[...] + d2_ref[...]
    w2 = jnp.exp(jnp.maximum(e2, e2 * 0.2))
    wex = jnp.dot(w2, r_ref[...], preferred_element_type=jnp.float32)
    den = acc_ref[0, :, DH:] + w2
    denx = jnp.dot(den, r_ref[...], preferred_element_type=jnp.float32)
    accs = jnp.concatenate([acc_ref[0, :, :DH], acc_ref[1, :, :DH]], axis=-1)
    acc = accs + h_ref[...] * wex
    o_ref[...] = acc / denx + b_ref[...] + x_ref[...]


def _tc_finalize(x, h, s2, d2, acc, R, bias2):
    blk = 2000
    grid = N // blk
    return pl.pallas_call(
        _fin_body,
        grid=(grid,),
        in_specs=[
            pl.BlockSpec((blk, D), lambda i: (i, 0)),
            pl.BlockSpec((blk, D), lambda i: (i, 0)),
            pl.BlockSpec((blk, 2 * H), lambda i: (i, 0)),
            pl.BlockSpec((blk, 2 * H), lambda i: (i, 0)),
            pl.BlockSpec((NC, blk, MW), lambda i: (0, i, 0)),
            pl.BlockSpec((2 * H, D), lambda i: (0, 0)),
            pl.BlockSpec((1, D), lambda i: (0, 0)),
        ],
        out_specs=pl.BlockSpec((blk, D), lambda i: (i, 0)),
        out_shape=jax.ShapeDtypeStruct((N, D), jnp.float32),
    )(x, h, s2, d2, acc, R, bias2)


def kernel(x, edge_index, W, att_src, att_dst, bias):
    src = edge_index[0].astype(jnp.int32)
    dst = edge_index[1].astype(jnp.int32)

    i = jnp.arange(D)
    j = jnp.arange(2 * H)
    sel = (i[:, None] // HD) == (j[None, :] % H)
    A_src2 = jnp.where(sel, att_src.reshape(D)[:, None], 0.0)
    A_dst2 = jnp.where(sel, att_dst.reshape(D)[:, None], 0.0)
    R = jnp.where((j[:, None] < H) & ((i[None, :] // HD) == j[:, None]),
                  1.0, 0.0)

    h, s2, d2 = _tc_prep(x, W, A_src2, A_dst2)
    h2 = jnp.stack([jnp.concatenate([h[:, :DH], s2], axis=1),
                    jnp.concatenate([h[:, DH:], s2], axis=1)])
    acc = _sc_edge(h2, d2, src, dst)
    return _tc_finalize(x, h, s2, d2, acc, R, bias[None, :])
